# Initial kernel scaffold; baseline (speedup 1.0000x reference)
import jax
import jax.numpy as jnp
from jax.experimental import pallas as pl


def kernel(dist, table):
    raise NotImplementedError("write your pallas kernel here")


# speedup vs baseline: 14.3256x; 14.3256x over previous
import jax
import jax.numpy as jnp
from jax.experimental import pallas as pl

_K = 24
_H = 16
_N = 512
_RB = 16


def _lookup_kernel(dist_ref, table_ref, out_ref):
    idx = dist_ref[...]
    iota = jax.lax.broadcasted_iota(jnp.int32, (_RB, _N, _K), 2)
    oh = (idx[:, :, None] == iota).astype(jnp.float32)
    ohm = oh.reshape(_RB * _N, _K)
    out_ref[...] = jnp.dot(ohm, table_ref[...],
                           preferred_element_type=jnp.float32)


def kernel(dist, table):
    B, N, _ = dist.shape
    table_eff = table.at[0].set(0.0)
    table_p = jnp.zeros((_K, _H), jnp.float32).at[:22, :].set(table_eff)

    d2 = dist.reshape(B * N, N)
    grid = (B * N) // _RB

    out = pl.pallas_call(
        _lookup_kernel,
        grid=(grid,),
        in_specs=[
            pl.BlockSpec((_RB, _N), lambda i: (i, 0)),
            pl.BlockSpec((_K, _H), lambda i: (0, 0)),
        ],
        out_specs=pl.BlockSpec((_RB * _N, _H), lambda i: (i, 0)),
        out_shape=jax.ShapeDtypeStruct((B * N * N, _H), jnp.float32),
    )(d2, table_p)
    return out.reshape(B, N, N, _H)


# speedup vs baseline: 14.3633x; 1.0026x over previous
import jax
import jax.numpy as jnp
from jax.experimental import pallas as pl

_K = 24
_H = 16
_N = 512
_RB = 16


def _lookup_kernel(dist_ref, table_ref, out_ref):
    idx = dist_ref[...]
    iota = jax.lax.broadcasted_iota(jnp.int32, (_RB, _N, _K), 2)
    oh = (idx[:, :, None] == iota).astype(jnp.float32)
    ohm = oh.reshape(_RB * _N, _K)
    res = jnp.dot(ohm, table_ref[...], preferred_element_type=jnp.float32)
    out_ref[...] = res.reshape(_RB, _N, _H)


def kernel(dist, table):
    B, N, _ = dist.shape
    table_eff = table.at[0].set(0.0)
    table_p = jnp.zeros((_K, _H), jnp.float32).at[:22, :].set(table_eff)

    d2 = dist.reshape(B * N, N)
    grid = (B * N) // _RB

    out = pl.pallas_call(
        _lookup_kernel,
        grid=(grid,),
        in_specs=[
            pl.BlockSpec((_RB, _N), lambda i: (i, 0)),
            pl.BlockSpec((_K, _H), lambda i: (0, 0)),
        ],
        out_specs=pl.BlockSpec((_RB, _N, _H), lambda i: (i, 0, 0)),
        out_shape=jax.ShapeDtypeStruct((B * N, N, _H), jnp.float32),
    )(d2, table_p)
    return out.reshape(B, N, N, _H)
